# Initial kernel scaffold; baseline (speedup 1.0000x reference)
import jax
import jax.numpy as jnp
from jax.experimental import pallas as pl


def kernel(logits, a, b):
    raise NotImplementedError("write your pallas kernel here")


# speedup vs baseline: 1.0804x; 1.0804x over previous
import functools

import jax
import jax.numpy as jnp
from jax import lax
from jax.experimental import pallas as pl
from jax.experimental.pallas import tpu as pltpu
from jax.experimental.pallas import tpu_sc as plsc

N = 100000
K = 16
M = 20


_LSE_ROWS = 1000


def _lse_body(x_ref, o_ref):
    x = x_ref[...]
    m = jnp.max(x, axis=1, keepdims=True)
    s = jnp.sum(jnp.exp(x - m), axis=1, keepdims=True)
    o_ref[...] = m + jnp.log(s)


def _lse(logits):
    n, k2 = logits.shape
    grid = n // _LSE_ROWS
    return pl.pallas_call(
        _lse_body,
        grid=(grid,),
        in_specs=[pl.BlockSpec((_LSE_ROWS, k2), lambda i: (i, 0))],
        out_specs=pl.BlockSpec((_LSE_ROWS, 1), lambda i: (i, 0)),
        out_shape=jax.ShapeDtypeStruct((n, 1), jnp.float32),
    )(logits)


_CHUNK_ROWS = 100
_CHUNK_ELEMS = _CHUNK_ROWS * M
_CHUNK_WORDS = _CHUNK_ROWS * 256
_NUM_CHUNKS = N // _CHUNK_ROWS
_NW = 32
_GROUPS = _CHUNK_ELEMS // 16


def _gather_sc(logits_flat, a_flat, b_flat):
    mesh = plsc.VectorSubcoreMesh(core_axis_name="c", subcore_axis_name="s")

    @functools.partial(
        pl.kernel,
        mesh=mesh,
        out_type=jax.ShapeDtypeStruct((N * M,), jnp.float32),
        compiler_params=pltpu.CompilerParams(needs_layout_passes=False),
        scratch_types=[
            pltpu.VMEM((_CHUNK_WORDS,), jnp.float32),
            pltpu.VMEM((_CHUNK_ELEMS,), jnp.int32),
            pltpu.VMEM((_CHUNK_ELEMS,), jnp.int32),
            pltpu.VMEM((_CHUNK_ELEMS,), jnp.float32),
        ],
    )
    def k(lg_hbm, a_hbm, b_hbm, out_hbm, rows_v, a_v, b_v, out_v):
        wid = lax.axis_index("s") * 2 + lax.axis_index("c")
        iota = lax.iota(jnp.int32, 16)

        def chunk_body(i, _):
            c = wid + i * _NW

            @pl.when(c < _NUM_CHUNKS)
            def _():
                pltpu.sync_copy(
                    lg_hbm.at[pl.ds(c * _CHUNK_WORDS, _CHUNK_WORDS)], rows_v)
                pltpu.sync_copy(
                    a_hbm.at[pl.ds(c * _CHUNK_ELEMS, _CHUNK_ELEMS)], a_v)
                pltpu.sync_copy(
                    b_hbm.at[pl.ds(c * _CHUNK_ELEMS, _CHUNK_ELEMS)], b_v)
                for g in range(_GROUPS):
                    base = g * 16
                    r = ((iota + base) * 3277) >> 16
                    av = a_v[pl.ds(base, 16)]
                    bv = b_v[pl.ds(base, 16)]
                    idx = (r << 8) + (av << 4) + bv
                    out_v[pl.ds(base, 16)] = plsc.load_gather(rows_v, [idx])
                pltpu.sync_copy(
                    out_v, out_hbm.at[pl.ds(c * _CHUNK_ELEMS, _CHUNK_ELEMS)])

            return _

        lax.fori_loop(0, 32, chunk_body, None)

    return k(logits_flat, a_flat, b_flat)


_CMB_ROWS = 1000


def _combine_body(g_ref, l_ref, o_ref):
    o_ref[...] = g_ref[...] - l_ref[...]


def _combine(g, lse):
    n = g.shape[0]
    grid = n // _CMB_ROWS
    return pl.pallas_call(
        _combine_body,
        grid=(grid,),
        in_specs=[
            pl.BlockSpec((_CMB_ROWS, M), lambda i: (i, 0)),
            pl.BlockSpec((_CMB_ROWS, 1), lambda i: (i, 0)),
        ],
        out_specs=pl.BlockSpec((_CMB_ROWS, M), lambda i: (i, 0)),
        out_shape=jax.ShapeDtypeStruct((n, M), jnp.float32),
    )(g, lse)


def kernel(logits, a, b):
    lse = _lse(logits)
    g = _gather_sc(
        logits.reshape(-1),
        a.astype(jnp.int32).reshape(-1),
        b.astype(jnp.int32).reshape(-1),
    )
    out = _combine(g.reshape(N, M), lse)
    return out.reshape(-1)
